# baseline (speedup 1.0000x reference)
import dataclasses
import functools

import jax
import jax.numpy as jnp
from jax import lax
from jax.experimental import pallas as pl
from jax.experimental.pallas import tpu as pltpu
from jax.experimental.pallas import tpu_sc as plsc

N = 10000
E = 320000
C = 128
NC = 2
NS = 16
NW = NC * NS
EPW = E // NW
K = 80
NCHUNK = EPW // K
NCHA = 128
EPWA = NCHA * K
EPAD = NW * EPWA

_mesh = plsc.VectorSubcoreMesh(core_axis_name="c", subcore_axis_name="s")

_cp_no_layout = pltpu.CompilerParams()
if "needs_layout_passes" in pltpu.CompilerParams.__dataclass_fields__:
    _cp_no_layout = dataclasses.replace(_cp_no_layout, needs_layout_passes=False)


@functools.partial(
    pl.kernel,
    out_type=jax.ShapeDtypeStruct((NW, N), jnp.float32),
    mesh=_mesh,
    scratch_types=[
        pltpu.VMEM((NCHUNK, K), jnp.int32),
        pltpu.VMEM((N,), jnp.float32),
    ],
    compiler_params=_cp_no_layout,
)
def _sc_degree(dst_hbm, degp_hbm, idx_v, deg_v):
    c = lax.axis_index("c")
    s = lax.axis_index("s")
    wid = c * NS + s
    pltpu.sync_copy(dst_hbm.at[wid], idx_v)

    zeros16 = jnp.zeros((16,), jnp.float32)

    @pl.loop(0, N // 16)
    def _(i):
        deg_v[pl.ds(i * 16, 16)] = zeros16

    ones16 = jnp.ones((16,), jnp.float32)

    @pl.loop(0, NCHUNK)
    def _(j):
        @pl.loop(0, K // 16)
        def _(l):
            idx16 = idx_v[j, pl.ds(l * 16, 16)]
            plsc.addupdate_scatter(deg_v, [idx16], ones16)

    pltpu.sync_copy(deg_v, degp_hbm.at[wid])


RPSA = 624
TBASE = RPSA * NS
TAIL = N - TBASE


def _rowwise_copy(s, src, dst):
    pltpu.sync_copy(src.at[pl.ds(s * RPSA, RPSA)], dst.at[pl.ds(s * RPSA, RPSA)])

    @pl.when(s == NS - 1)
    def _():
        pltpu.sync_copy(src.at[pl.ds(TBASE, TAIL)], dst.at[pl.ds(TBASE, TAIL)])


@functools.partial(
    pl.kernel,
    out_type=jax.ShapeDtypeStruct((NC, N, C), jnp.float32),
    mesh=_mesh,
    scratch_types=[
        pltpu.VMEM((2, K), jnp.int32),
        pltpu.VMEM((2, K), jnp.int32),
        pltpu.VMEM((2, K), jnp.int32),
        pltpu.VMEM((2, K), jnp.int32),
        pltpu.VMEM((K, C), jnp.float32),
        pltpu.VMEM((K, C), jnp.float32),
        pltpu.VMEM_SHARED((N, C), jnp.float32),
        pltpu.SemaphoreType.DMA,
        pltpu.SemaphoreType.DMA,
        pltpu.SemaphoreType.DMA,
        pltpu.SemaphoreType.DMA,
        pltpu.SemaphoreType.DMA,
        pltpu.SemaphoreType.DMA,
    ],
)
def _sc_aggregate(edges_hbm, z_hbm, h2p_hbm, acc_hbm,
                  i0, i1, i2, i3, ra, rb, acc_sh,
                  si0, si1, si2, si3, sa, sb):
    c = lax.axis_index("c")
    s = lax.axis_index("s")
    wid = c * NS + s
    _rowwise_copy(s, z_hbm, acc_sh)

    idx = (i0, i1, i2, i3)
    isem = (si0, si1, si2, si3)
    row = (ra, rb)
    gsem = (sa, sb)

    for t in range(4):
        pltpu.make_async_copy(edges_hbm.at[wid, t], idx[t], isem[t]).start()
    plsc.subcore_barrier()
    for t in range(2):
        pltpu.make_async_copy(edges_hbm.at[wid, t], idx[t], isem[t]).wait()
        pltpu.make_async_copy(h2p_hbm.at[idx[t].at[0]], row[t], gsem[t]).start()

    @pl.loop(0, NCHA // 4)
    def _(j):
        base = 4 * j
        for t in range(4):
            cc = base + t
            n4 = jnp.where(cc + 4 >= NCHA, cc + 4 - NCHA, cc + 4)
            n2 = jnp.where(cc + 2 >= NCHA, cc + 2 - NCHA, cc + 2)
            ib, sI = idx[t], isem[t]
            ig, sg = idx[(t + 2) % 4], isem[(t + 2) % 4]
            rbuf, sG = row[t % 2], gsem[t % 2]
            pltpu.make_async_copy(h2p_hbm.at[ib.at[0]], rbuf, sG).wait()
            pltpu.sync_copy(rbuf, acc_sh.at[ib.at[1]], add=True)
            pltpu.make_async_copy(edges_hbm.at[wid, n4], ib, sI).start()
            pltpu.make_async_copy(edges_hbm.at[wid, n2], ig, sg).wait()
            pltpu.make_async_copy(h2p_hbm.at[ig.at[0]], rbuf, sG).start()

    for t in range(2):
        pltpu.make_async_copy(h2p_hbm.at[idx[t].at[0]], row[t], gsem[t]).wait()
    for t in (2, 3):
        pltpu.make_async_copy(edges_hbm.at[wid, t], idx[t], isem[t]).wait()

    plsc.subcore_barrier()
    _rowwise_copy(s, acc_sh, acc_hbm.at[c])


BM = 1000


def _deg_col(degp_ref):
    return jnp.sum(degp_ref[...], axis=1, keepdims=True) + 1.0


def _mmps_body(degp_ref, x_ref, w_ref, h2_ref):
    h = jnp.dot(x_ref[...], w_ref[...], preferred_element_type=jnp.float32)
    h2_ref[...] = lax.rsqrt(_deg_col(degp_ref)) * h


def _final_body(degp_ref, acc_ref, h2_ref, b_ref, o_ref):
    dis = lax.rsqrt(_deg_col(degp_ref))
    tot = acc_ref[0] + acc_ref[1] + h2_ref[...]
    o_ref[...] = jnp.maximum(dis * tot + b_ref[...], 0.0)


def kernel(x, edge_index, W, b):
    srcf = edge_index[0].astype(jnp.int32)
    dstf = edge_index[1].astype(jnp.int32)
    dst = dstf.reshape(NW, NCHUNK, K)
    srcp = jnp.concatenate([srcf, jnp.full((EPAD - E,), N, jnp.int32)])
    dstp = jnp.concatenate([dstf, jnp.zeros((EPAD - E,), jnp.int32)])
    edges = jnp.stack([srcp.reshape(NW, NCHA, K),
                       dstp.reshape(NW, NCHA, K)], axis=2)
    zfull = jnp.zeros((N, C), jnp.float32)

    degp = _sc_degree(dst).T

    h2 = pl.pallas_call(
        _mmps_body,
        grid=(N // BM,),
        in_specs=[
            pl.BlockSpec((BM, NW), lambda i: (i, 0)),
            pl.BlockSpec((BM, C), lambda i: (i, 0)),
            pl.BlockSpec((C, C), lambda i: (0, 0)),
        ],
        out_specs=pl.BlockSpec((BM, C), lambda i: (i, 0)),
        out_shape=jax.ShapeDtypeStruct((N, C), jnp.float32),
    )(degp, x, W)

    h2p = jnp.concatenate([h2, jnp.zeros((8, C), jnp.float32)])
    acc = _sc_aggregate(edges, zfull, h2p)

    out = pl.pallas_call(
        _final_body,
        grid=(N // BM,),
        in_specs=[
            pl.BlockSpec((BM, NW), lambda i: (i, 0)),
            pl.BlockSpec((NC, BM, C), lambda i: (0, i, 0)),
            pl.BlockSpec((BM, C), lambda i: (i, 0)),
            pl.BlockSpec((1, C), lambda i: (0, 0)),
        ],
        out_specs=pl.BlockSpec((BM, C), lambda i: (i, 0)),
        out_shape=jax.ShapeDtypeStruct((N, C), jnp.float32),
    )(degp, acc, h2, b.reshape(1, C))

    return out


# speedup vs baseline: 2.5762x; 2.5762x over previous
import dataclasses
import functools

import jax
import jax.numpy as jnp
from jax import lax
from jax.experimental import pallas as pl
from jax.experimental.pallas import tpu as pltpu
from jax.experimental.pallas import tpu_sc as plsc

N = 10000
E = 320000
C = 128
NC = 2
NS = 16
NW = NC * NS
EPW = E // NW
K = 80
NCHUNK = EPW // K
NCHA = 128
EPWA = NCHA * K
EPAD = NW * EPWA

_mesh = plsc.VectorSubcoreMesh(core_axis_name="c", subcore_axis_name="s")

_cp_no_layout = pltpu.CompilerParams()
if "needs_layout_passes" in pltpu.CompilerParams.__dataclass_fields__:
    _cp_no_layout = dataclasses.replace(_cp_no_layout, needs_layout_passes=False)


@functools.partial(
    pl.kernel,
    out_type=jax.ShapeDtypeStruct((NW, N), jnp.float32),
    mesh=_mesh,
    scratch_types=[
        pltpu.VMEM((NCHUNK, K), jnp.int32),
        pltpu.VMEM((N,), jnp.float32),
    ],
    compiler_params=_cp_no_layout,
)
def _sc_degree(dst_hbm, degp_hbm, idx_v, deg_v):
    c = lax.axis_index("c")
    s = lax.axis_index("s")
    wid = c * NS + s
    pltpu.sync_copy(dst_hbm.at[wid], idx_v)

    zeros16 = jnp.zeros((16,), jnp.float32)

    @pl.loop(0, N // 16)
    def _(i):
        deg_v[pl.ds(i * 16, 16)] = zeros16

    ones16 = jnp.ones((16,), jnp.float32)

    @pl.loop(0, NCHUNK)
    def _(j):
        @pl.loop(0, K // 16)
        def _(l):
            idx16 = idx_v[j, pl.ds(l * 16, 16)]
            plsc.addupdate_scatter(deg_v, [idx16], ones16)

    pltpu.sync_copy(deg_v, degp_hbm.at[wid])


RPSA = 624
TBASE = RPSA * NS
TAIL = N - TBASE


def _rowwise_copy(s, src, dst):
    pltpu.sync_copy(src.at[pl.ds(s * RPSA, RPSA)], dst.at[pl.ds(s * RPSA, RPSA)])

    @pl.when(s == NS - 1)
    def _():
        pltpu.sync_copy(src.at[pl.ds(TBASE, TAIL)], dst.at[pl.ds(TBASE, TAIL)])


@functools.partial(
    pl.kernel,
    out_type=jax.ShapeDtypeStruct((NC, N, C), jnp.float32),
    mesh=_mesh,
    scratch_types=[
        pltpu.VMEM((2, K), jnp.int32),
        pltpu.VMEM((2, K), jnp.int32),
        pltpu.VMEM((2, K), jnp.int32),
        pltpu.VMEM((2, K), jnp.int32),
        pltpu.VMEM((K, C), jnp.float32),
        pltpu.VMEM((K, C), jnp.float32),
        pltpu.VMEM_SHARED((N, C), jnp.float32),
        pltpu.SemaphoreType.DMA,
        pltpu.SemaphoreType.DMA,
        pltpu.SemaphoreType.DMA,
        pltpu.SemaphoreType.DMA,
        pltpu.SemaphoreType.DMA,
        pltpu.SemaphoreType.DMA,
    ],
)
def _sc_aggregate(edges_hbm, z_hbm, h2p_hbm, acc_hbm,
                  i0, i1, i2, i3, ra, rb, acc_sh,
                  si0, si1, si2, si3, sa, sb):
    c = lax.axis_index("c")
    s = lax.axis_index("s")
    wid = c * NS + s
    _rowwise_copy(s, z_hbm, acc_sh)

    idx = (i0, i1, i2, i3)
    isem = (si0, si1, si2, si3)
    row = (ra, rb)
    gsem = (sa, sb)

    for t in range(4):
        pltpu.make_async_copy(edges_hbm.at[wid, t], idx[t], isem[t]).start()
    plsc.subcore_barrier()
    for t in range(2):
        pltpu.make_async_copy(edges_hbm.at[wid, t], idx[t], isem[t]).wait()
        pltpu.make_async_copy(h2p_hbm.at[idx[t].at[0]], row[t], gsem[t]).start()

    @pl.loop(0, NCHA // 4)
    def _(j):
        base = 4 * j
        for t in range(4):
            cc = base + t
            n4 = jnp.where(cc + 4 >= NCHA, cc + 4 - NCHA, cc + 4)
            n2 = jnp.where(cc + 2 >= NCHA, cc + 2 - NCHA, cc + 2)
            ib, sI = idx[t], isem[t]
            ig, sg = idx[(t + 2) % 4], isem[(t + 2) % 4]
            rbuf, sG = row[t % 2], gsem[t % 2]
            pltpu.make_async_copy(h2p_hbm.at[ib.at[0]], rbuf, sG).wait()
            pltpu.sync_copy(rbuf, acc_sh.at[ib.at[1]], add=True)
            pltpu.make_async_copy(edges_hbm.at[wid, n4], ib, sI).start()
            pltpu.make_async_copy(edges_hbm.at[wid, n2], ig, sg).wait()
            pltpu.make_async_copy(h2p_hbm.at[ig.at[0]], rbuf, sG).start()

    for t in range(2):
        pltpu.make_async_copy(h2p_hbm.at[idx[t].at[0]], row[t], gsem[t]).wait()
    for t in (2, 3):
        pltpu.make_async_copy(edges_hbm.at[wid, t], idx[t], isem[t]).wait()

    plsc.subcore_barrier()
    _rowwise_copy(s, acc_sh, acc_hbm.at[c])


BM = 1000


def _deg_col(degp_ref):
    return jnp.sum(degp_ref[...], axis=1, keepdims=True) + 1.0


def _mmps_body(degp_ref, x_ref, w_ref, h2_ref):
    h = jnp.dot(x_ref[...], w_ref[...], preferred_element_type=jnp.float32)
    h2_ref[...] = lax.rsqrt(_deg_col(degp_ref)) * h


def _final_body(degp_ref, acc_ref, h2_ref, b_ref, o_ref):
    dis = lax.rsqrt(_deg_col(degp_ref))
    tot = acc_ref[0] + acc_ref[1] + h2_ref[...]
    o_ref[...] = jnp.maximum(dis * tot + b_ref[...], 0.0)


def kernel(x, edge_index, W, b):
    srcf = edge_index[0].astype(jnp.int32)
    dstf = edge_index[1].astype(jnp.int32)
    dst = dstf.reshape(NW, NCHUNK, K)
    pad_iota = jnp.arange(EPAD - E, dtype=jnp.int32)
    srcp = jnp.concatenate([srcf, N + (pad_iota % 8)])
    dstp = jnp.concatenate([dstf, pad_iota % N])
    edges = jnp.stack([srcp.reshape(NW, NCHA, K),
                       dstp.reshape(NW, NCHA, K)], axis=2)
    zfull = jnp.zeros((N, C), jnp.float32)

    degp = _sc_degree(dst).T

    h2 = pl.pallas_call(
        _mmps_body,
        grid=(N // BM,),
        in_specs=[
            pl.BlockSpec((BM, NW), lambda i: (i, 0)),
            pl.BlockSpec((BM, C), lambda i: (i, 0)),
            pl.BlockSpec((C, C), lambda i: (0, 0)),
        ],
        out_specs=pl.BlockSpec((BM, C), lambda i: (i, 0)),
        out_shape=jax.ShapeDtypeStruct((N, C), jnp.float32),
    )(degp, x, W)

    h2p = jnp.concatenate([h2, jnp.zeros((8, C), jnp.float32)])
    acc = _sc_aggregate(edges, zfull, h2p)

    out = pl.pallas_call(
        _final_body,
        grid=(N // BM,),
        in_specs=[
            pl.BlockSpec((BM, NW), lambda i: (i, 0)),
            pl.BlockSpec((NC, BM, C), lambda i: (0, i, 0)),
            pl.BlockSpec((BM, C), lambda i: (i, 0)),
            pl.BlockSpec((1, C), lambda i: (0, 0)),
        ],
        out_specs=pl.BlockSpec((BM, C), lambda i: (i, 0)),
        out_shape=jax.ShapeDtypeStruct((N, C), jnp.float32),
    )(degp, acc, h2, b.reshape(1, C))

    return out
